# baseline (speedup 1.0000x reference)
import functools
import jax
import jax.numpy as jnp
from jax.experimental import pallas as pl
from jax.experimental.pallas import tpu as pltpu


def _router_kernel(x_ref, w1_ref, b1_ref, w2_ref, b2_ref, w3_ref, b3_ref,
                   g_ref, h1s_ref, lg_ref, run_ref,
                   *, E, capacity, A, Bn, Cn, bt, bk):
    i = pl.program_id(0)

    @pl.when(i == 0)
    def _():
        lg_ref[...] = jnp.zeros_like(lg_ref)
        run_ref[...] = jnp.zeros_like(run_ref)

    @pl.when(i < A)
    def _():
        h1 = jnp.dot(x_ref[...], w1_ref[...], preferred_element_type=jnp.float32)
        h1s_ref[i] = jnp.maximum(h1 + b1_ref[...], 0.0)

    @pl.when((i >= A) & (i < A + Bn))
    def _():
        w2 = w2_ref[...]
        acc = jnp.dot(h1s_ref[0], w2[0:bk, :], preferred_element_type=jnp.float32)
        for j in range(1, A):
            acc += jnp.dot(h1s_ref[j], w2[j * bk:(j + 1) * bk, :],
                           preferred_element_type=jnp.float32)
        h2 = jnp.maximum(acc + b2_ref[...], 0.0)
        lg_ref[...] += jnp.dot(h2, w3_ref[...], preferred_element_type=jnp.float32)

    @pl.when(i >= A + Bn)
    def _():
        ic = i - (A + Bn)
        logits = (lg_ref[pl.ds(ic * bt, bt), :] + b3_ref[...])[:, :E]
        m = jnp.max(logits, axis=1, keepdims=True)
        ex = jnp.exp(logits - m)
        p = ex / jnp.sum(ex, axis=1, keepdims=True)
        iota = jax.lax.broadcasted_iota(jnp.int32, (bt, E), 1)
        m0 = jnp.max(p, axis=1, keepdims=True)
        e0 = jnp.min(jnp.where(p == m0, iota, E), axis=1, keepdims=True)
        oh0 = iota == e0
        pm = jnp.where(oh0, -jnp.inf, p)
        m1 = jnp.max(pm, axis=1, keepdims=True)
        e1 = jnp.min(jnp.where(pm == m1, iota, E), axis=1, keepdims=True)
        oh1 = iota == e1
        c = oh0.astype(jnp.float32) + oh1.astype(jnp.float32)
        r = jax.lax.broadcasted_iota(jnp.int32, (bt, bt), 0)
        cc = jax.lax.broadcasted_iota(jnp.int32, (bt, bt), 1)
        tri = (cc < r).astype(jnp.float32)
        pos = jnp.dot(tri, c, preferred_element_type=jnp.float32) + run_ref[...]
        p0 = jnp.sum(jnp.where(oh0, pos, 0.0), axis=1, keepdims=True)
        p1 = jnp.sum(jnp.where(oh1, pos, 0.0), axis=1, keepdims=True)
        keep0 = (p0 < capacity).astype(jnp.float32)
        keep1 = (p1 < capacity).astype(jnp.float32)
        g_ref[...] = m0 * keep0 + m1 * keep1
        run_ref[...] += jnp.sum(c, axis=0, keepdims=True)


def _router_gate(x, W1, b1, W2, b2, W3p, b3p, E, capacity):
    M, C = x.shape
    H = W1.shape[1]
    NL = W3p.shape[1]
    A = 8
    bk = H // A
    bn2 = 128
    Bn = H // bn2
    bt = 128
    Cn = M // bt
    nsteps = A + Bn + Cn
    return pl.pallas_call(
        functools.partial(_router_kernel, E=E, capacity=capacity,
                          A=A, Bn=Bn, Cn=Cn, bt=bt, bk=bk),
        grid=(nsteps,),
        in_specs=[
            pl.BlockSpec((M, C), lambda i: (0, 0)),
            pl.BlockSpec((C, bk), lambda i: (0, jnp.minimum(i, A - 1))),
            pl.BlockSpec((1, bk), lambda i: (0, jnp.minimum(i, A - 1))),
            pl.BlockSpec((H, bn2), lambda i: (0, jnp.clip(i - A, 0, Bn - 1))),
            pl.BlockSpec((1, bn2), lambda i: (0, jnp.clip(i - A, 0, Bn - 1))),
            pl.BlockSpec((bn2, NL), lambda i: (jnp.clip(i - A, 0, Bn - 1), 0)),
            pl.BlockSpec((1, NL), lambda i: (0, 0)),
        ],
        out_specs=pl.BlockSpec(
            (bt, 1), lambda i: (jnp.clip(i - (A + Bn), 0, Cn - 1), 0)),
        out_shape=jax.ShapeDtypeStruct((M, 1), jnp.float32),
        scratch_shapes=[
            pltpu.VMEM((A, M, bk), jnp.float32),
            pltpu.VMEM((M, NL), jnp.float32),
            pltpu.VMEM((1, E), jnp.float32),
        ],
        compiler_params=pltpu.CompilerParams(
            dimension_semantics=("arbitrary",)),
    )(x, W1, b1.reshape(1, H), W2, b2.reshape(1, H), W3p, b3p.reshape(1, NL))


def _ffn_kernel(x_ref, w1_ref, b1_ref, w2_ref, b2_ref, g_ref, o_ref):
    y1 = jnp.dot(x_ref[...], w1_ref[...], preferred_element_type=jnp.float32)
    y1 = jnp.maximum(y1 + b1_ref[...], 0.0)
    y = jnp.dot(y1, w2_ref[...], preferred_element_type=jnp.float32)
    o_ref[...] = (y + b2_ref[...]) * g_ref[...]


def _ffn_gate(x, We1, be1, We2, be2, gate, bm):
    M, C = x.shape
    _, H = We1.shape
    _, N = We2.shape
    return pl.pallas_call(
        _ffn_kernel,
        grid=(M // bm,),
        in_specs=[
            pl.BlockSpec((bm, C), lambda m: (m, 0)),
            pl.BlockSpec((C, H), lambda m: (0, 0)),
            pl.BlockSpec((1, H), lambda m: (0, 0)),
            pl.BlockSpec((H, N), lambda m: (0, 0)),
            pl.BlockSpec((1, N), lambda m: (0, 0)),
            pl.BlockSpec((bm, 1), lambda m: (m, 0)),
        ],
        out_specs=pl.BlockSpec((bm, N), lambda m: (m, 0)),
        out_shape=jax.ShapeDtypeStruct((M, N), jnp.float32),
    )(x, We1, be1.reshape(1, H), We2, be2.reshape(1, N), gate)


def kernel(x, W1, b1, W2, b2, W3, b3, We1, be1, We2, be2):
    B, T, C = x.shape
    E = W3.shape[1]
    capacity = int(T / E * 1.25)
    xf = x.reshape(T, C)

    W3p = jnp.pad(W3, ((0, 0), (0, 128 - E)))
    b3p = jnp.pad(b3, (0, 128 - E))
    gate = _router_gate(xf, W1, b1, W2, b2, W3p, b3p, E, capacity)
    out = _ffn_gate(xf, We1, be1, We2, be2, gate, bm=512)
    return out.reshape(B, T, C)


# speedup vs baseline: 1.3091x; 1.3091x over previous
import functools
import jax
import jax.numpy as jnp
from jax.experimental import pallas as pl
from jax.experimental.pallas import tpu as pltpu


def _mm_bias_kernel(a_ref, w_ref, b_ref, o_ref, *, relu):
    acc = jnp.dot(a_ref[...], w_ref[...], preferred_element_type=jnp.float32)
    acc = acc + b_ref[...]
    if relu:
        acc = jnp.maximum(acc, 0.0)
    o_ref[...] = acc


def _mm_bias(a, w, b, relu, bm, bn):
    M, K = a.shape
    _, N = w.shape
    return pl.pallas_call(
        functools.partial(_mm_bias_kernel, relu=relu),
        grid=(N // bn, M // bm),
        in_specs=[
            pl.BlockSpec((bm, K), lambda n, m: (m, 0)),
            pl.BlockSpec((K, bn), lambda n, m: (0, n)),
            pl.BlockSpec((1, bn), lambda n, m: (0, n)),
        ],
        out_specs=pl.BlockSpec((bm, bn), lambda n, m: (m, n)),
        out_shape=jax.ShapeDtypeStruct((M, N), jnp.float32),
    )(a, w, b.reshape(1, N))


def _h2lg_kernel(h1_ref, w2_ref, b2_ref, w3_ref, b3_ref,
                 g_ref, lg_ref, run_ref,
                 *, E, capacity, Bn, Cn, bt):
    i = pl.program_id(0)

    @pl.when(i == 0)
    def _():
        lg_ref[...] = jnp.zeros_like(lg_ref)
        run_ref[...] = jnp.zeros_like(run_ref)

    @pl.when(i < Bn)
    def _():
        h2 = jnp.dot(h1_ref[...], w2_ref[...], preferred_element_type=jnp.float32)
        h2 = jnp.maximum(h2 + b2_ref[...], 0.0)
        lg_ref[...] += jnp.dot(h2, w3_ref[...], preferred_element_type=jnp.float32)

    @pl.when(i >= Bn)
    def _():
        ic = i - Bn
        logits = (lg_ref[pl.ds(ic * bt, bt), :] + b3_ref[...])[:, :E]
        m = jnp.max(logits, axis=1, keepdims=True)
        ex = jnp.exp(logits - m)
        p = ex / jnp.sum(ex, axis=1, keepdims=True)
        iota = jax.lax.broadcasted_iota(jnp.int32, (bt, E), 1)
        m0 = jnp.max(p, axis=1, keepdims=True)
        e0 = jnp.min(jnp.where(p == m0, iota, E), axis=1, keepdims=True)
        oh0 = iota == e0
        pm = jnp.where(oh0, -jnp.inf, p)
        m1 = jnp.max(pm, axis=1, keepdims=True)
        e1 = jnp.min(jnp.where(pm == m1, iota, E), axis=1, keepdims=True)
        oh1 = iota == e1
        c = oh0.astype(jnp.float32) + oh1.astype(jnp.float32)
        r = jax.lax.broadcasted_iota(jnp.int32, (bt, bt), 0)
        cc = jax.lax.broadcasted_iota(jnp.int32, (bt, bt), 1)
        tri = (cc < r).astype(jnp.float32)
        pos = jnp.dot(tri, c, preferred_element_type=jnp.float32) + run_ref[...]
        p0 = jnp.sum(jnp.where(oh0, pos, 0.0), axis=1, keepdims=True)
        p1 = jnp.sum(jnp.where(oh1, pos, 0.0), axis=1, keepdims=True)
        keep0 = (p0 < capacity).astype(jnp.float32)
        keep1 = (p1 < capacity).astype(jnp.float32)
        g_ref[...] = m0 * keep0 + m1 * keep1
        run_ref[...] += jnp.sum(c, axis=0, keepdims=True)


def _h2lg_gate(h1, W2, b2, W3p, b3p, E, capacity):
    M, H = h1.shape
    NL = W3p.shape[1]
    bn2 = 256
    Bn = H // bn2
    bt = 128
    Cn = M // bt
    return pl.pallas_call(
        functools.partial(_h2lg_kernel, E=E, capacity=capacity,
                          Bn=Bn, Cn=Cn, bt=bt),
        grid=(Bn + Cn,),
        in_specs=[
            pl.BlockSpec((M, H), lambda i: (0, 0)),
            pl.BlockSpec((H, bn2), lambda i: (0, jnp.clip(i, 0, Bn - 1))),
            pl.BlockSpec((1, bn2), lambda i: (0, jnp.clip(i, 0, Bn - 1))),
            pl.BlockSpec((bn2, NL), lambda i: (jnp.clip(i, 0, Bn - 1), 0)),
            pl.BlockSpec((1, NL), lambda i: (0, 0)),
        ],
        out_specs=pl.BlockSpec(
            (bt, 1), lambda i: (jnp.clip(i - Bn, 0, Cn - 1), 0)),
        out_shape=jax.ShapeDtypeStruct((M, 1), jnp.float32),
        scratch_shapes=[
            pltpu.VMEM((M, NL), jnp.float32),
            pltpu.VMEM((1, E), jnp.float32),
        ],
        compiler_params=pltpu.CompilerParams(
            dimension_semantics=("arbitrary",)),
    )(h1, W2, b2.reshape(1, H), W3p, b3p.reshape(1, NL))


def _ffn_kernel(x_ref, w1_ref, b1_ref, w2_ref, b2_ref, g_ref, o_ref):
    y1 = jnp.dot(x_ref[...], w1_ref[...], preferred_element_type=jnp.float32)
    y1 = jnp.maximum(y1 + b1_ref[...], 0.0)
    y = jnp.dot(y1, w2_ref[...], preferred_element_type=jnp.float32)
    o_ref[...] = (y + b2_ref[...]) * g_ref[...]


def _ffn_gate(x, We1, be1, We2, be2, gate, bm):
    M, C = x.shape
    _, H = We1.shape
    _, N = We2.shape
    return pl.pallas_call(
        _ffn_kernel,
        grid=(M // bm,),
        in_specs=[
            pl.BlockSpec((bm, C), lambda m: (m, 0)),
            pl.BlockSpec((C, H), lambda m: (0, 0)),
            pl.BlockSpec((1, H), lambda m: (0, 0)),
            pl.BlockSpec((H, N), lambda m: (0, 0)),
            pl.BlockSpec((1, N), lambda m: (0, 0)),
            pl.BlockSpec((bm, 1), lambda m: (m, 0)),
        ],
        out_specs=pl.BlockSpec((bm, N), lambda m: (m, 0)),
        out_shape=jax.ShapeDtypeStruct((M, N), jnp.float32),
    )(x, We1, be1.reshape(1, H), We2, be2.reshape(1, N), gate)


def kernel(x, W1, b1, W2, b2, W3, b3, We1, be1, We2, be2):
    B, T, C = x.shape
    E = W3.shape[1]
    capacity = int(T / E * 1.25)
    xf = x.reshape(T, C)

    W3p = jnp.pad(W3, ((0, 0), (0, 128 - E)))
    b3p = jnp.pad(b3, (0, 128 - E))
    h1 = _mm_bias(xf, W1, b1, relu=True, bm=2048, bn=512)
    gate = _h2lg_gate(h1, W2, b2, W3p, b3p, E, capacity)
    out = _ffn_gate(xf, We1, be1, We2, be2, gate, bm=512)
    return out.reshape(B, T, C)
